# baseline (speedup 1.0000x reference)
import functools

import jax
import jax.numpy as jnp
from jax import lax
from jax.experimental import pallas as pl
from jax.experimental.pallas import tpu as pltpu
from jax.experimental.pallas import tpu_sc as plsc

N_NODES = 10000
N_EDGES = 320000
D = 128
DH = D // 2

NC = 2
NS = 16
NW = NC * NS
CHUNK = 128
K0 = 88
K1 = 72
HK1 = K0 // 2
NROWS = NS * (K0 + K1) + 16
EP = NROWS * CHUNK
ACC_ROWS = 10112
RPW = ACC_ROWS // NS
LAST = N_NODES - (NS - 1) * RPW

_sc_mesh = plsc.VectorSubcoreMesh(core_axis_name="c", subcore_axis_name="s")


@functools.partial(
    pl.kernel,
    out_type=jax.ShapeDtypeStruct((NC, N_NODES, D), jnp.float32),
    mesh=_sc_mesh,
    compiler_params=pltpu.CompilerParams(use_tc_tiling_on_sc=False),
    scratch_types=[
        pltpu.MemorySpace.VMEM_SHARED((ACC_ROWS, D), jnp.float32),
        pltpu.VMEM((HK1, CHUNK), jnp.int32),
        pltpu.VMEM((HK1, CHUNK), jnp.int32),
        pltpu.VMEM((2, CHUNK, DH), jnp.int32),
        pltpu.VMEM((2, CHUNK // 2, D), jnp.float32),
        pltpu.SemaphoreType.DMA,
        pltpu.SemaphoreType.DMA,
    ],
)
def _sc_aggregate(packed_hbm, src_hbm, dst_hbm, zero_hbm, out_hbm,
                  acc, src_v, dst_v, pkd, rf32, gsem, ssem):
    c = lax.axis_index("c")
    s = lax.axis_index("s")
    row0 = jnp.where(c == 0, s * K0, NS * K0 + s * K1)
    hkc = jnp.where(c == 0, K0 // 2, K1 // 2)
    npairs = jnp.where(c == 0, K0 // 4, K1 // 4)

    pltpu.sync_copy(zero_hbm, acc.at[pl.ds(s * RPW, RPW)])
    plsc.subcore_barrier()

    hi_mask = jnp.full((16,), -65536, jnp.int32)
    HC = CHUNK // 2

    def unpack_half(b, h):
        def row_body(r, carry):
            for c4 in range(DH // 16):
                x = pkd[b, HC * h + r, pl.ds(16 * c4, 16)]
                lo = lax.bitcast_convert_type(lax.shift_left(x, 16), jnp.float32)
                hi = lax.bitcast_convert_type(lax.bitwise_and(x, hi_mask),
                                              jnp.float32)
                rf32[h, r, pl.ds(16 * c4, 16)] = lo
                rf32[h, r, pl.ds(16 * c4 + 64, 16)] = hi
            return carry

        lax.fori_loop(0, HC, row_body, 0)

    def wait_scatter(h):
        pltpu.make_async_copy(rf32.at[h], acc.at[dst_v.at[0, pl.ds(0, HC)]],
                              ssem).wait()

    for phase in range(2):
        base = row0 + phase * hkc
        pltpu.sync_copy(src_hbm.at[pl.ds(base, HK1)], src_v)
        pltpu.sync_copy(dst_hbm.at[pl.ds(base, HK1)], dst_v)
        pltpu.async_copy(packed_hbm.at[src_v.at[0]], pkd.at[0], gsem)

        def pair_body(j2, carry):
            for b in range(2):
                j = 2 * j2 + b
                pltpu.make_async_copy(packed_hbm.at[src_v.at[j]],
                                      pkd.at[b], gsem).wait()

                @pl.when(j + 1 < hkc)
                def _():
                    pltpu.async_copy(packed_hbm.at[src_v.at[j + 1]],
                                     pkd.at[1 - b], gsem)

                for h in range(2):
                    if phase == 0:
                        @pl.when(j > 0)
                        def _():
                            wait_scatter(h)
                    else:
                        wait_scatter(h)
                    unpack_half(b, h)
                    pltpu.async_copy(rf32.at[h],
                                     acc.at[dst_v.at[j, pl.ds(HC * h, HC)]],
                                     ssem, add=True)
            return carry

        lax.fori_loop(0, npairs, pair_body, 0)

    wait_scatter(0)
    wait_scatter(1)
    plsc.subcore_barrier()

    @pl.when(s < NS - 1)
    def _():
        pltpu.sync_copy(acc.at[pl.ds(s * RPW, RPW)],
                        out_hbm.at[c, pl.ds(s * RPW, RPW)])

    @pl.when(s == NS - 1)
    def _():
        pltpu.sync_copy(acc.at[pl.ds((NS - 1) * RPW, LAST)],
                        out_hbm.at[c, pl.ds((NS - 1) * RPW, LAST)])


def _tc_linear_body(p_ref, w_ref, b_ref, o_ref):
    x = p_ref[0] + p_ref[1]
    y = lax.dot_general(x, w_ref[...], (((1,), (1,)), ((), ())),
                        preferred_element_type=jnp.float32)
    o_ref[...] = y + b_ref[0:1, :]


def _tc_linear(parts, W, b8):
    M = 1000
    return pl.pallas_call(
        _tc_linear_body,
        grid=(N_NODES // M,),
        in_specs=[
            pl.BlockSpec((NC, M, D), lambda i: (0, i, 0)),
            pl.BlockSpec((D, D), lambda i: (0, 0)),
            pl.BlockSpec((8, D), lambda i: (0, 0)),
        ],
        out_specs=pl.BlockSpec((M, D), lambda i: (i, 0)),
        out_shape=jax.ShapeDtypeStruct((N_NODES, D), jnp.float32),
    )(parts, W, b8)


def kernel(feature, edge_index, W, b):
    src = edge_index[0].astype(jnp.int32)
    dst = edge_index[1].astype(jnp.int32)
    pad = EP - N_EDGES
    src_p = jnp.concatenate([src, jnp.zeros((pad,), jnp.int32)]).reshape(NROWS, CHUNK)
    dst_p = jnp.concatenate([dst, jnp.full((pad,), N_NODES, jnp.int32)]).reshape(NROWS, CHUNK)
    zeros = jnp.zeros((RPW, D), jnp.float32)
    fb = feature.astype(jnp.bfloat16)
    lo = lax.bitcast_convert_type(fb[:, :DH], jnp.uint16).astype(jnp.uint32)
    hi = lax.bitcast_convert_type(fb[:, DH:], jnp.uint16).astype(jnp.uint32)
    packed = lax.bitcast_convert_type(lo | (hi << 16), jnp.int32)
    parts = _sc_aggregate(packed, src_p, dst_p, zeros)
    return _tc_linear(parts, W, jnp.broadcast_to(b, (8, D)))


# speedup vs baseline: 1.0424x; 1.0424x over previous
import functools

import jax
import jax.numpy as jnp
from jax import lax
from jax.experimental import pallas as pl
from jax.experimental.pallas import tpu as pltpu
from jax.experimental.pallas import tpu_sc as plsc

N_NODES = 10000
N_EDGES = 320000
D = 128
DH = D // 2

NC = 2
NS = 16
NW = NC * NS
CHUNK = 128
K0 = 96
K1 = 64
HK1 = K0 // 2
NROWS = NS * (K0 + K1) + 16
EP = NROWS * CHUNK
ACC_ROWS = 10112
RPW = ACC_ROWS // NS
LAST = N_NODES - (NS - 1) * RPW

_sc_mesh = plsc.VectorSubcoreMesh(core_axis_name="c", subcore_axis_name="s")


@functools.partial(
    pl.kernel,
    out_type=jax.ShapeDtypeStruct((NC, N_NODES, D), jnp.float32),
    mesh=_sc_mesh,
    compiler_params=pltpu.CompilerParams(use_tc_tiling_on_sc=False),
    scratch_types=[
        pltpu.MemorySpace.VMEM_SHARED((ACC_ROWS, D), jnp.float32),
        pltpu.VMEM((HK1, CHUNK), jnp.int32),
        pltpu.VMEM((HK1, CHUNK), jnp.int32),
        pltpu.VMEM((2, CHUNK, DH), jnp.int32),
        pltpu.VMEM((2, CHUNK // 2, D), jnp.float32),
        pltpu.SemaphoreType.DMA,
        pltpu.SemaphoreType.DMA,
    ],
)
def _sc_aggregate(packed_hbm, src_hbm, dst_hbm, zero_hbm, out_hbm,
                  acc, src_v, dst_v, pkd, rf32, gsem, ssem):
    c = lax.axis_index("c")
    s = lax.axis_index("s")
    row0 = jnp.where(c == 0, s * K0, NS * K0 + s * K1)
    hkc = jnp.where(c == 0, K0 // 2, K1 // 2)
    npairs = jnp.where(c == 0, K0 // 4, K1 // 4)

    pltpu.sync_copy(zero_hbm, acc.at[pl.ds(s * RPW, RPW)])
    plsc.subcore_barrier()

    hi_mask = jnp.full((16,), -65536, jnp.int32)
    HC = CHUNK // 2

    def unpack_half(b, h):
        def row_body(r, carry):
            for c4 in range(DH // 16):
                x = pkd[b, HC * h + r, pl.ds(16 * c4, 16)]
                lo = lax.bitcast_convert_type(lax.shift_left(x, 16), jnp.float32)
                hi = lax.bitcast_convert_type(lax.bitwise_and(x, hi_mask),
                                              jnp.float32)
                rf32[h, r, pl.ds(16 * c4, 16)] = lo
                rf32[h, r, pl.ds(16 * c4 + 64, 16)] = hi
            return carry

        lax.fori_loop(0, HC, row_body, 0)

    def wait_scatter(h):
        pltpu.make_async_copy(rf32.at[h], acc.at[dst_v.at[0, pl.ds(0, HC)]],
                              ssem).wait()

    for phase in range(2):
        base = row0 + phase * hkc
        pltpu.sync_copy(src_hbm.at[pl.ds(base, HK1)], src_v)
        pltpu.sync_copy(dst_hbm.at[pl.ds(base, HK1)], dst_v)
        pltpu.async_copy(packed_hbm.at[src_v.at[0]], pkd.at[0], gsem)

        def pair_body(j2, carry):
            for b in range(2):
                j = 2 * j2 + b
                pltpu.make_async_copy(packed_hbm.at[src_v.at[j]],
                                      pkd.at[b], gsem).wait()

                @pl.when(j + 1 < hkc)
                def _():
                    pltpu.async_copy(packed_hbm.at[src_v.at[j + 1]],
                                     pkd.at[1 - b], gsem)

                for h in range(2):
                    if phase == 0:
                        @pl.when(j > 0)
                        def _():
                            wait_scatter(h)
                    else:
                        wait_scatter(h)
                    unpack_half(b, h)
                    pltpu.async_copy(rf32.at[h],
                                     acc.at[dst_v.at[j, pl.ds(HC * h, HC)]],
                                     ssem, add=True)
            return carry

        lax.fori_loop(0, npairs, pair_body, 0)

    wait_scatter(0)
    wait_scatter(1)
    plsc.subcore_barrier()

    @pl.when(s < NS - 1)
    def _():
        pltpu.sync_copy(acc.at[pl.ds(s * RPW, RPW)],
                        out_hbm.at[c, pl.ds(s * RPW, RPW)])

    @pl.when(s == NS - 1)
    def _():
        pltpu.sync_copy(acc.at[pl.ds((NS - 1) * RPW, LAST)],
                        out_hbm.at[c, pl.ds((NS - 1) * RPW, LAST)])


def _tc_linear_body(p_ref, w_ref, b_ref, o_ref):
    x = p_ref[0] + p_ref[1]
    y = lax.dot_general(x, w_ref[...], (((1,), (1,)), ((), ())),
                        preferred_element_type=jnp.float32)
    o_ref[...] = y + b_ref[0:1, :]


def _tc_linear(parts, W, b8):
    M = 1000
    return pl.pallas_call(
        _tc_linear_body,
        grid=(N_NODES // M,),
        in_specs=[
            pl.BlockSpec((NC, M, D), lambda i: (0, i, 0)),
            pl.BlockSpec((D, D), lambda i: (0, 0)),
            pl.BlockSpec((8, D), lambda i: (0, 0)),
        ],
        out_specs=pl.BlockSpec((M, D), lambda i: (i, 0)),
        out_shape=jax.ShapeDtypeStruct((N_NODES, D), jnp.float32),
    )(parts, W, b8)


def kernel(feature, edge_index, W, b):
    src = edge_index[0].astype(jnp.int32)
    dst = edge_index[1].astype(jnp.int32)
    pad = EP - N_EDGES
    src_p = jnp.concatenate([src, jnp.zeros((pad,), jnp.int32)]).reshape(NROWS, CHUNK)
    dst_p = jnp.concatenate([dst, jnp.full((pad,), N_NODES, jnp.int32)]).reshape(NROWS, CHUNK)
    zeros = jnp.zeros((RPW, D), jnp.float32)
    fb = feature.astype(jnp.bfloat16)
    lo = lax.bitcast_convert_type(fb[:, :DH], jnp.uint16).astype(jnp.uint32)
    hi = lax.bitcast_convert_type(fb[:, DH:], jnp.uint16).astype(jnp.uint32)
    packed = lax.bitcast_convert_type(lo | (hi << 16), jnp.int32)
    parts = _sc_aggregate(packed, src_p, dst_p, zeros)
    return _tc_linear(parts, W, jnp.broadcast_to(b, (8, D)))
